# baseline (speedup 1.0000x reference)
import functools

import jax
import jax.numpy as jnp
from jax import lax
from jax.experimental import pallas as pl
from jax.experimental.pallas import tpu as pltpu
from jax.experimental.pallas import tpu_sc as plsc

_NC = 2
_NS = 16
_NW = _NC * _NS

_P = 50
_TI = 128
_NTT = 128
_D = 64
_CT = _D // 8
_TPW = _NTT // _NW
_NE = 1000000
_UNITS = _TPW * _P


_RB = 2048


def _tc_prep_body(wt_ref, out_ref):
    x = wt_ref[...]
    eye = jnp.eye(_D, dtype=jnp.float32)
    xt = jax.lax.dot_general(x, eye, (((0,), (0,)), ((), ())),
                             preferred_element_type=jnp.float32)
    out_ref[:, 0:_D] = xt
    out_ref[:, _D:] = xt


def _make_tc_prep():
    grid = (_NE + _RB - 1) // _RB
    return pl.pallas_call(
        _tc_prep_body,
        grid=(grid,),
        in_specs=[pl.BlockSpec((_D, _RB), lambda i: (0, i))],
        out_specs=pl.BlockSpec((_RB, 2 * _D), lambda i: (i, 0)),
        out_shape=jax.ShapeDtypeStruct((_NE, 2 * _D), jnp.float32),
    )


def _make_gather():
    mesh = plsc.VectorSubcoreMesh(core_axis_name="c", subcore_axis_name="s")

    @functools.partial(
        pl.kernel,
        out_type=jax.ShapeDtypeStruct((_P, _CT, _NTT, 8, _TI), jnp.float32),
        mesh=mesh,
        scratch_types=[
            pltpu.VMEM((_TPW * _TI * _P,), jnp.int32),
            pltpu.VMEM((_P, _TPW * _TI), jnp.int32),
            pltpu.VMEM((2, _TI, _D), jnp.float32),
            pltpu.VMEM((2, _CT, 1, 8, _TI + 1), jnp.float32),
            pltpu.SemaphoreType.DMA,
            pltpu.SemaphoreType.DMA,
            pltpu.SemaphoreType.DMA,
            pltpu.SemaphoreType.DMA,
        ],
        compiler_params=pltpu.CompilerParams(use_tc_tiling_on_sc=False,
                                             needs_layout_passes=False),
    )
    def gather_kernel(tid_hbm, table_hbm, out_hbm, idx_raw, idx_v, rows_v,
                      obuf_v, sg0, sg1, sw0, sw1):
        semg = (sg0, sg1)
        semw = (sw0, sw1)
        wid = lax.axis_index("s") * _NC + lax.axis_index("c")
        tt0 = wid * _TPW
        nblk = _TPW * _TI * _P

        lanes = lax.iota(jnp.int32, 16)
        c_ids = [lanes + (16 * cg) for cg in range(_D // 16)]
        ct_ids = [c // 8 for c in c_ids]
        ci_ids = [c - (c // 8) * 8 for c in c_ids]
        zeros16 = jnp.full((16,), 0, jnp.int32)

        pltpu.sync_copy(
            tid_hbm.at[pl.ds(pl.multiple_of(tt0 * _TI * _P, nblk), nblk)],
            idx_raw,
        )
        lanes_p = lanes * _P

        def idxt_body(p, carry):
            def grp_body(k, carry2):
                addr = lanes_p + (k * (16 * _P) + p)
                v = plsc.load_gather(idx_raw, [addr])
                idx_v[p, pl.ds(k * 16, 16)] = v + v
                return carry2
            lax.fori_loop(0, _TPW * _TI // 16, grp_body, carry)
            return carry
        lax.fori_loop(0, _P, idxt_body, 0)

        def unit_parts(u):
            tti = u // _P
            return tti, u - tti * _P

        def fire_gather(u, b):
            tti, p = unit_parts(u)
            pltpu.async_copy(
                table_hbm.at[idx_v.at[p, pl.ds(pl.multiple_of(tti * _TI, _TI),
                                               _TI)]],
                rows_v.at[b],
                semg[b],
            )

        def drain_gather(b):
            pltpu.make_async_copy(
                table_hbm.at[pl.ds(0, _TI)], rows_v.at[b], semg[b]
            ).wait()


        def transpose(b):
            bsplat = jnp.full((16,), b, jnp.int32)

            def tok_body(t, carry):
                tsplat = zeros16 + t
                for cg in range(_D // 16):
                    v = rows_v[b, t, pl.ds(cg * 16, 16)]
                    plsc.store_scatter(
                        obuf_v,
                        [bsplat, ct_ids[cg], zeros16, ci_ids[cg], tsplat],
                        v)
                return carry
            lax.fori_loop(0, _TI, tok_body, 0, unroll=8)

        def fire_writes(u, b):
            tti, p = unit_parts(u)
            tt = tt0 + tti
            pltpu.async_copy(
                obuf_v.at[b].at[:, :, :, pl.ds(0, _TI)],
                out_hbm.at[p, pl.ds(0, _CT), pl.ds(tt, 1)],
                semw[b],
            )

        def wait_writes(b):
            pltpu.make_async_copy(
                obuf_v.at[b].at[:, :, :, pl.ds(0, _TI)],
                out_hbm.at[0, pl.ds(0, _CT), pl.ds(0, 1)],
                semw[b],
            ).wait()

        fire_gather(0, 0)

        @pl.loop(0, _UNITS, step=2)
        def _(u0):
            for bb in range(2):
                u = u0 + bb
                nb = 1 - bb

                @pl.when(u + 1 < _UNITS)
                def _():
                    fire_gather(u + 1, nb)

                drain_gather(bb)

                @pl.when(u0 >= 2 - bb)
                def _():
                    wait_writes(bb)

                transpose(bb)
                fire_writes(u, bb)

        wait_writes(0)
        wait_writes(1)

    return gather_kernel


def kernel(token_ids, W):
    tid = token_ids.reshape(-1).astype(jnp.int32)
    Wp = _make_tc_prep()(W.T).reshape(2 * _NE, _D)
    ot = _make_gather()(tid, Wp)
    return ot.transpose(2, 4, 0, 1, 3).reshape(16384, _P, _D)


# speedup vs baseline: 1.0603x; 1.0603x over previous
import functools

import jax
import jax.numpy as jnp
from jax import lax
from jax.experimental import pallas as pl
from jax.experimental.pallas import tpu as pltpu
from jax.experimental.pallas import tpu_sc as plsc

_NC = 2
_NS = 16
_NW = _NC * _NS

_P = 50
_TI = 128
_NTT = 128
_D = 64
_CT = _D // 8
_TPW = _NTT // _NW
_NE = 1000000
_UNITS = _TPW * _P


_RB = 2048


def _tc_prep_body(wt_ref, out_ref):
    x = wt_ref[...]
    out_ref[:, 0:_D] = jnp.swapaxes(x, 0, 1)
    out_ref[:, _D:] = jnp.zeros((_RB, _D), jnp.float32)


def _make_tc_prep():
    grid = (_NE + _RB - 1) // _RB
    return pl.pallas_call(
        _tc_prep_body,
        grid=(grid,),
        in_specs=[pl.BlockSpec((_D, _RB), lambda i: (0, i))],
        out_specs=pl.BlockSpec((_RB, 2 * _D), lambda i: (i, 0)),
        out_shape=jax.ShapeDtypeStruct((_NE, 2 * _D), jnp.float32),
    )


def _make_gather():
    mesh = plsc.VectorSubcoreMesh(core_axis_name="c", subcore_axis_name="s")

    @functools.partial(
        pl.kernel,
        out_type=jax.ShapeDtypeStruct((_P, _CT, _NTT, 8, _TI), jnp.float32),
        mesh=mesh,
        scratch_types=[
            pltpu.VMEM((_TPW * _TI * _P,), jnp.int32),
            pltpu.VMEM((_P, _TPW * _TI), jnp.int32),
            pltpu.VMEM((2, _TI, _D), jnp.float32),
            pltpu.VMEM((2, _CT, 1, 8, _TI + 1), jnp.float32),
            pltpu.SemaphoreType.DMA,
            pltpu.SemaphoreType.DMA,
            pltpu.SemaphoreType.DMA,
            pltpu.SemaphoreType.DMA,
        ],
        compiler_params=pltpu.CompilerParams(use_tc_tiling_on_sc=False,
                                             needs_layout_passes=False),
    )
    def gather_kernel(tid_hbm, table_hbm, out_hbm, idx_raw, idx_v, rows_v,
                      obuf_v, sg0, sg1, sw0, sw1):
        semg = (sg0, sg1)
        semw = (sw0, sw1)
        wid = lax.axis_index("s") * _NC + lax.axis_index("c")
        tt0 = wid * _TPW
        nblk = _TPW * _TI * _P

        lanes = lax.iota(jnp.int32, 16)
        c_ids = [lanes + (16 * cg) for cg in range(_D // 16)]
        ct_ids = [c // 8 for c in c_ids]
        ci_ids = [c - (c // 8) * 8 for c in c_ids]
        zeros16 = jnp.full((16,), 0, jnp.int32)

        pltpu.sync_copy(
            tid_hbm.at[pl.ds(pl.multiple_of(tt0 * _TI * _P, nblk), nblk)],
            idx_raw,
        )
        lanes_p = lanes * _P

        def idxt_body(p, carry):
            def grp_body(k, carry2):
                addr = lanes_p + (k * (16 * _P) + p)
                v = plsc.load_gather(idx_raw, [addr])
                idx_v[p, pl.ds(k * 16, 16)] = v + v
                return carry2
            lax.fori_loop(0, _TPW * _TI // 16, grp_body, carry)
            return carry
        lax.fori_loop(0, _P, idxt_body, 0)

        def unit_parts(u):
            tti = u // _P
            return tti, u - tti * _P

        def fire_gather(u, b):
            tti, p = unit_parts(u)
            pltpu.async_copy(
                table_hbm.at[idx_v.at[p, pl.ds(pl.multiple_of(tti * _TI, _TI),
                                               _TI)]],
                rows_v.at[b],
                semg[b],
            )

        def drain_gather(b):
            pltpu.make_async_copy(
                table_hbm.at[pl.ds(0, _TI)], rows_v.at[b], semg[b]
            ).wait()


        def transpose(b):
            bsplat = jnp.full((16,), b, jnp.int32)

            def tok_body(t, carry):
                tsplat = zeros16 + t
                for cg in range(_D // 16):
                    v = rows_v[b, t, pl.ds(cg * 16, 16)]
                    plsc.store_scatter(
                        obuf_v,
                        [bsplat, ct_ids[cg], zeros16, ci_ids[cg], tsplat],
                        v)
                return carry
            lax.fori_loop(0, _TI, tok_body, 0, unroll=8)

        def fire_writes(u, b):
            tti, p = unit_parts(u)
            tt = tt0 + tti
            pltpu.async_copy(
                obuf_v.at[b].at[:, :, :, pl.ds(0, _TI)],
                out_hbm.at[p, pl.ds(0, _CT), pl.ds(tt, 1)],
                semw[b],
            )

        def wait_writes(b):
            pltpu.make_async_copy(
                obuf_v.at[b].at[:, :, :, pl.ds(0, _TI)],
                out_hbm.at[0, pl.ds(0, _CT), pl.ds(0, 1)],
                semw[b],
            ).wait()

        fire_gather(0, 0)

        @pl.loop(0, _UNITS, step=2)
        def _(u0):
            for bb in range(2):
                u = u0 + bb
                nb = 1 - bb

                @pl.when(u + 1 < _UNITS)
                def _():
                    fire_gather(u + 1, nb)

                drain_gather(bb)

                @pl.when(u0 >= 2 - bb)
                def _():
                    wait_writes(bb)

                transpose(bb)
                fire_writes(u, bb)

        wait_writes(0)
        wait_writes(1)

    return gather_kernel


def kernel(token_ids, W):
    tid = token_ids.reshape(-1).astype(jnp.int32)
    Wp = _make_tc_prep()(W.T).reshape(2 * _NE, _D)
    ot = _make_gather()(tid, Wp)
    return ot.transpose(2, 4, 0, 1, 3).reshape(16384, _P, _D)
